# Initial kernel scaffold; baseline (speedup 1.0000x reference)
import jax
import jax.numpy as jnp
from jax.experimental import pallas as pl


def kernel(x, W):
    raise NotImplementedError("write your pallas kernel here")


# speedup vs baseline: 1.9723x; 1.9723x over previous
import jax
import jax.numpy as jnp
from jax.experimental import pallas as pl
from jax.experimental.pallas import tpu as pltpu

NUM_EXPERTS = 16
TOP_K = 2
AUX_LOSS_COEF = 0.01
Z_LOSS_COEF = 0.001


def _router_block(x_ref, w_ref, idx_ref, wgt_ref, aux_ref,
                  cnt_acc, p_acc, z_acc, *, n_tokens):
    step = pl.program_id(0)
    nsteps = pl.num_programs(0)

    @pl.when(step == 0)
    def _init():
        cnt_acc[...] = jnp.zeros_like(cnt_acc)
        p_acc[...] = jnp.zeros_like(p_acc)
        z_acc[...] = jnp.zeros_like(z_acc)

    x = x_ref[...]
    w = w_ref[...]
    logits = jax.lax.dot_general(x, w, (((1,), (1,)), ((), ())))

    m = jnp.max(logits, axis=1, keepdims=True)
    e = jnp.exp(logits - m)
    s = jnp.sum(e, axis=1, keepdims=True)
    weights = e / s

    lse = m + jnp.log(s)
    z_acc[...] += jnp.sum(lse * lse).reshape(1, 1)
    p_acc[...] += jnp.sum(weights, axis=0, keepdims=True)

    iota = jax.lax.broadcasted_iota(jnp.int32, weights.shape, 1)
    w1 = jnp.max(weights, axis=1, keepdims=True)
    i1 = jnp.min(jnp.where(weights == w1, iota, NUM_EXPERTS),
                 axis=1, keepdims=True)
    masked = jnp.where(iota == i1, -jnp.inf, weights)
    w2 = jnp.max(masked, axis=1, keepdims=True)
    i2 = jnp.min(jnp.where(masked == w2, iota, NUM_EXPERTS),
                 axis=1, keepdims=True)

    onehot = ((iota == i1) | (iota == i2)).astype(jnp.float32)
    cnt_acc[...] += jnp.sum(onehot, axis=0, keepdims=True)

    tot = w1 + w2
    idx_ref[:, 0:1] = i1
    idx_ref[:, 1:2] = i2
    wgt_ref[:, 0:1] = w1 / tot
    wgt_ref[:, 1:2] = w2 / tot

    @pl.when(step == nsteps - 1)
    def _fin():
        f = cnt_acc[...] / (n_tokens * TOP_K)
        p = p_acc[...] / n_tokens
        balance = NUM_EXPERTS * jnp.sum(f * p)
        z = z_acc[...] / n_tokens
        aux_ref[...] = (AUX_LOSS_COEF * balance
                        + Z_LOSS_COEF * z).reshape(1, 1)


def kernel(x, W):
    import functools
    b, s, d = x.shape
    n = b * s
    xf = x.reshape(n, d)
    T = 512
    grid = (n // T,)
    idx, wgt, aux = pl.pallas_call(
        functools.partial(_router_block, n_tokens=n),
        grid=grid,
        in_specs=[
            pl.BlockSpec((T, d), lambda i: (i, 0)),
            pl.BlockSpec((NUM_EXPERTS, d), lambda i: (0, 0)),
        ],
        out_specs=[
            pl.BlockSpec((T, TOP_K), lambda i: (i, 0)),
            pl.BlockSpec((T, TOP_K), lambda i: (i, 0)),
            pl.BlockSpec((1, 1), lambda i: (0, 0)),
        ],
        out_shape=[
            jax.ShapeDtypeStruct((n, TOP_K), jnp.int32),
            jax.ShapeDtypeStruct((n, TOP_K), jnp.float32),
            jax.ShapeDtypeStruct((1, 1), jnp.float32),
        ],
        scratch_shapes=[
            pltpu.VMEM((1, NUM_EXPERTS), jnp.float32),
            pltpu.VMEM((1, NUM_EXPERTS), jnp.float32),
            pltpu.VMEM((1, 1), jnp.float32),
        ],
    )(xf, W)
    return (idx.reshape(b, s, TOP_K), wgt.reshape(b, s, TOP_K),
            aux.reshape(()))


# speedup vs baseline: 2.2825x; 1.1573x over previous
import jax
import jax.numpy as jnp
from jax.experimental import pallas as pl
from jax.experimental.pallas import tpu as pltpu

NUM_EXPERTS = 16
TOP_K = 2
AUX_LOSS_COEF = 0.01
Z_LOSS_COEF = 0.001


def _router_block(x_ref, w_ref, idx_ref, wgt_ref, aux_ref,
                  cnt_acc, p_acc, z_acc, *, n_tokens):
    step = pl.program_id(0)
    nsteps = pl.num_programs(0)

    @pl.when(step == 0)
    def _init():
        cnt_acc[...] = jnp.zeros_like(cnt_acc)
        p_acc[...] = jnp.zeros_like(p_acc)
        z_acc[...] = jnp.zeros_like(z_acc)

    x = x_ref[...]
    w = w_ref[...]
    logits = jax.lax.dot_general(x, w, (((1,), (1,)), ((), ())))

    m = jnp.max(logits, axis=1, keepdims=True)
    e = jnp.exp(logits - m)
    s = jnp.sum(e, axis=1, keepdims=True)
    weights = e / s

    lse = m + jnp.log(s)
    z_acc[...] += jnp.sum(lse * lse).reshape(1, 1)
    p_acc[...] += jnp.sum(weights, axis=0, keepdims=True)

    iota = jax.lax.broadcasted_iota(jnp.int32, weights.shape, 1)
    w1 = jnp.max(weights, axis=1, keepdims=True)
    i1 = jnp.min(jnp.where(weights == w1, iota, NUM_EXPERTS),
                 axis=1, keepdims=True)
    masked = jnp.where(iota == i1, -jnp.inf, weights)
    w2 = jnp.max(masked, axis=1, keepdims=True)
    i2 = jnp.min(jnp.where(masked == w2, iota, NUM_EXPERTS),
                 axis=1, keepdims=True)

    onehot = ((iota == i1) | (iota == i2)).astype(jnp.float32)
    cnt_acc[...] += jnp.sum(onehot, axis=0, keepdims=True)

    tot = w1 + w2
    idx_ref[:, 0:1] = i1
    idx_ref[:, 1:2] = i2
    wgt_ref[:, 0:1] = w1 / tot
    wgt_ref[:, 1:2] = w2 / tot

    @pl.when(step == nsteps - 1)
    def _fin():
        f = cnt_acc[...] / (n_tokens * TOP_K)
        p = p_acc[...] / n_tokens
        balance = NUM_EXPERTS * jnp.sum(f * p)
        z = z_acc[...] / n_tokens
        aux_ref[...] = (AUX_LOSS_COEF * balance
                        + Z_LOSS_COEF * z).reshape(1, 1)


def kernel(x, W):
    import functools
    b, s, d = x.shape
    n = b * s
    xf = x.reshape(n, d)
    T = 1024
    grid = (n // T,)
    idx, wgt, aux = pl.pallas_call(
        functools.partial(_router_block, n_tokens=n),
        grid=grid,
        in_specs=[
            pl.BlockSpec((T, d), lambda i: (i, 0)),
            pl.BlockSpec((NUM_EXPERTS, d), lambda i: (0, 0)),
        ],
        out_specs=[
            pl.BlockSpec((T, TOP_K), lambda i: (i, 0)),
            pl.BlockSpec((T, TOP_K), lambda i: (i, 0)),
            pl.BlockSpec((1, 1), lambda i: (0, 0)),
        ],
        out_shape=[
            jax.ShapeDtypeStruct((n, TOP_K), jnp.int32),
            jax.ShapeDtypeStruct((n, TOP_K), jnp.float32),
            jax.ShapeDtypeStruct((1, 1), jnp.float32),
        ],
        scratch_shapes=[
            pltpu.VMEM((1, NUM_EXPERTS), jnp.float32),
            pltpu.VMEM((1, NUM_EXPERTS), jnp.float32),
            pltpu.VMEM((1, 1), jnp.float32),
        ],
    )(xf, W)
    return (idx.reshape(b, s, TOP_K), wgt.reshape(b, s, TOP_K),
            aux.reshape(()))


# speedup vs baseline: 2.3018x; 1.0084x over previous
import jax
import jax.numpy as jnp
from jax.experimental import pallas as pl
from jax.experimental.pallas import tpu as pltpu

NUM_EXPERTS = 16
TOP_K = 2
AUX_LOSS_COEF = 0.01
Z_LOSS_COEF = 0.001


def _router_block(x_ref, w_ref, idx_ref, wgt_ref, aux_ref,
                  cnt_acc, p_acc, z_acc, *, n_tokens):
    step = pl.program_id(0)
    nsteps = pl.num_programs(0)

    @pl.when(step == 0)
    def _init():
        cnt_acc[...] = jnp.zeros_like(cnt_acc)
        p_acc[...] = jnp.zeros_like(p_acc)
        z_acc[...] = jnp.zeros_like(z_acc)

    x = x_ref[...]
    w = w_ref[...]
    logits = jax.lax.dot_general(x, w, (((1,), (1,)), ((), ())))

    m = jnp.max(logits, axis=1, keepdims=True)
    e = jnp.exp(logits - m)
    s = jnp.sum(e, axis=1, keepdims=True)
    weights = e / s

    lse = m + jnp.log(s)
    z_acc[...] += jnp.sum(lse * lse).reshape(1, 1)
    p_acc[...] += jnp.sum(weights, axis=0, keepdims=True)

    iota = jax.lax.broadcasted_iota(jnp.int32, weights.shape, 1)
    w1 = jnp.max(weights, axis=1, keepdims=True)
    i1 = jnp.min(jnp.where(weights == w1, iota, NUM_EXPERTS),
                 axis=1, keepdims=True)
    masked = jnp.where(iota == i1, -jnp.inf, weights)
    w2 = jnp.max(masked, axis=1, keepdims=True)
    i2 = jnp.min(jnp.where(masked == w2, iota, NUM_EXPERTS),
                 axis=1, keepdims=True)

    onehot = ((iota == i1) | (iota == i2)).astype(jnp.float32)
    cnt_acc[...] += jnp.sum(onehot, axis=0, keepdims=True)

    tot = w1 + w2
    idx_ref[:, 0:1] = i1
    idx_ref[:, 1:2] = i2
    wgt_ref[:, 0:1] = w1 / tot
    wgt_ref[:, 1:2] = w2 / tot

    @pl.when(step == nsteps - 1)
    def _fin():
        f = cnt_acc[...] / (n_tokens * TOP_K)
        p = p_acc[...] / n_tokens
        balance = NUM_EXPERTS * jnp.sum(f * p)
        z = z_acc[...] / n_tokens
        aux_ref[...] = (AUX_LOSS_COEF * balance
                        + Z_LOSS_COEF * z).reshape(1, 1)


def kernel(x, W):
    import functools
    b, s, d = x.shape
    n = b * s
    xf = x.reshape(n, d)
    T = 2048
    grid = (n // T,)
    idx, wgt, aux = pl.pallas_call(
        functools.partial(_router_block, n_tokens=n),
        grid=grid,
        in_specs=[
            pl.BlockSpec((T, d), lambda i: (i, 0)),
            pl.BlockSpec((NUM_EXPERTS, d), lambda i: (0, 0)),
        ],
        out_specs=[
            pl.BlockSpec((T, TOP_K), lambda i: (i, 0)),
            pl.BlockSpec((T, TOP_K), lambda i: (i, 0)),
            pl.BlockSpec((1, 1), lambda i: (0, 0)),
        ],
        out_shape=[
            jax.ShapeDtypeStruct((n, TOP_K), jnp.int32),
            jax.ShapeDtypeStruct((n, TOP_K), jnp.float32),
            jax.ShapeDtypeStruct((1, 1), jnp.float32),
        ],
        scratch_shapes=[
            pltpu.VMEM((1, NUM_EXPERTS), jnp.float32),
            pltpu.VMEM((1, NUM_EXPERTS), jnp.float32),
            pltpu.VMEM((1, 1), jnp.float32),
        ],
    )(xf, W)
    return (idx.reshape(b, s, TOP_K), wgt.reshape(b, s, TOP_K),
            aux.reshape(()))
